# baseline (speedup 1.0000x reference)
import functools

import jax
import jax.numpy as jnp
from jax import lax
from jax.experimental import pallas as pl
from jax.experimental.pallas import tpu as pltpu
from jax.experimental.pallas import tpu_sc as plsc

HIDDEN = 128
DEPTH = 6
ATOM_FDIM = 35
BOND_FDIM = 5
MAX_NB = 10
N_ATOMS = 10000
N_BONDS = 320000
N_MESS = 4096
N_MOLS = 100
ATOMS_PER_MOL = 100
N_TBL = N_MESS + N_BONDS

NW = 32
LANES = 16
_PREC = jax.lax.Precision.HIGHEST

BM = 512


def _gather_sum_sc(tbl, idx_arr, B, CH):
    bpw = B // NW
    nch = bpw // CH
    assert nch % 2 == 0
    mesh = plsc.VectorSubcoreMesh(core_axis_name="c", subcore_axis_name="s")

    @functools.partial(
        pl.kernel,
        mesh=mesh,
        out_type=jax.ShapeDtypeStruct((B, HIDDEN), jnp.float32),
    scratch_types=[
            pltpu.VMEM((2, MAX_NB, CH), jnp.int32),
            pltpu.VMEM((2, MAX_NB, CH, HIDDEN), jnp.float32),
            pltpu.VMEM((2, CH, HIDDEN), jnp.float32),
            pltpu.SemaphoreType.DMA,
            pltpu.SemaphoreType.DMA,
            pltpu.SemaphoreType.DMA,
            pltpu.SemaphoreType.DMA,
        ],
    )
    def k(tbl_hbm, idx_hbm, out_hbm, idx_v, rows_v, acc_v, sem0, sem1, os0, os1):
        wid = lax.axis_index("s") * 2 + lax.axis_index("c")
        sems = (sem0, sem1)
        osems = (os0, os1)

        H2 = CH // 2

        def fetch(c, buf):
            pltpu.sync_copy(idx_hbm.at[wid * nch + c], idx_v.at[buf])
            for r in range(MAX_NB):
                for h in range(2):
                    pltpu.async_copy(
                        tbl_hbm.at[idx_v.at[buf, r, pl.ds(H2 * h, H2)]],
                        rows_v.at[buf, r, pl.ds(H2 * h, H2)], sems[buf])

        def drain(c, buf):
            for r in range(MAX_NB):
                for h in range(2):
                    pltpu.make_async_copy(
                        tbl_hbm.at[idx_v.at[buf, r, pl.ds(H2 * h, H2)]],
                        rows_v.at[buf, r, pl.ds(H2 * h, H2)],
                        sems[buf]).wait()

        def out_slice(c):
            return out_hbm.at[pl.ds(wid * bpw + c * CH, CH)]

        def accum(c, buf):
            @plsc.parallel_loop(0, CH, 1, unroll=2)
            def seg(i):
                for j in range(HIDDEN // LANES):
                    sl = pl.ds(LANES * j, LANES)
                    s = rows_v[buf, 0, i, sl]
                    for r in range(1, MAX_NB):
                        s = s + rows_v[buf, r, i, sl]
                    acc_v[buf, i, sl] = s

            pltpu.async_copy(acc_v.at[buf], out_slice(c), osems[buf])

        def wait_store(c, buf):
            pltpu.make_async_copy(acc_v.at[buf], out_slice(c), osems[buf]).wait()

        fetch(0, 0)

        def body(i, carry):
            c0 = 2 * i
            fetch(c0 + 1, 1)
            drain(c0, 0)

            @pl.when(i > 0)
            def _():
                wait_store(c0 - 2, 0)

            accum(c0, 0)

            @pl.when(c0 + 2 < nch)
            def _():
                fetch(c0 + 2, 0)

            drain(c0 + 1, 1)

            @pl.when(i > 0)
            def _():
                wait_store(c0 - 1, 1)

            accum(c0 + 1, 1)
            return carry

        lax.fori_loop(0, nch // 2, body, 0)
        wait_store(nch - 2, 0)
        wait_store(nch - 1, 1)

    return k(tbl, idx_arr)


def _arrange_idx(g, B, CH):
    nch = B // NW // CH
    g = g.astype(jnp.int32).reshape(NW * nch, CH, MAX_NB)
    return jnp.transpose(g, (0, 2, 1))


def _tree_transform(tree, W_h, Wo2):
    def body(t_ref, wh_ref, wo_ref, th_ref, to_ref):
        t = t_ref[...]
        th_ref[...] = jnp.dot(t, wh_ref[...], precision=_PREC)
        to_ref[...] = jnp.dot(t, wo_ref[...], precision=_PREC)

    nb = N_MESS // BM
    return pl.pallas_call(
        body,
        grid=(nb,),
        in_specs=[
            pl.BlockSpec((BM, HIDDEN), lambda i: (i, 0)),
            pl.BlockSpec((HIDDEN, HIDDEN), lambda i: (0, 0)),
            pl.BlockSpec((HIDDEN, HIDDEN), lambda i: (0, 0)),
        ],
        out_specs=(
            pl.BlockSpec((BM, HIDDEN), lambda i: (i, 0)),
            pl.BlockSpec((BM, HIDDEN), lambda i: (i, 0)),
        ),
        out_shape=(
            jax.ShapeDtypeStruct((N_TBL, HIDDEN), jnp.float32),
            jax.ShapeDtypeStruct((N_TBL, HIDDEN), jnp.float32),
        ),
    )(tree, W_h, Wo2)


def _bond_init(tblH0, fbp, Wip, W_h):
    def body(tbl_ref, fb_ref, wi_ref, wh_ref, bi_ref, gh_ref):
        bi = jnp.dot(fb_ref[...], wi_ref[...], precision=_PREC)
        bi_ref[...] = bi
        gh_ref[...] = jnp.dot(jnp.maximum(bi, 0.0), wh_ref[...], precision=_PREC)

    nb = N_BONDS // BM
    off = N_MESS // BM
    return pl.pallas_call(
        body,
        grid=(nb,),
        in_specs=[
            pl.BlockSpec(memory_space=pl.ANY),
            pl.BlockSpec((BM, 64), lambda i: (i, 0)),
            pl.BlockSpec((64, HIDDEN), lambda i: (0, 0)),
            pl.BlockSpec((HIDDEN, HIDDEN), lambda i: (0, 0)),
        ],
        out_specs=(
            pl.BlockSpec((BM, HIDDEN), lambda i: (i, 0)),
            pl.BlockSpec((BM, HIDDEN), lambda i: (i + off, 0)),
        ),
        out_shape=(
            jax.ShapeDtypeStruct((N_BONDS, HIDDEN), jnp.float32),
            jax.ShapeDtypeStruct((N_TBL, HIDDEN), jnp.float32),
        ),
        input_output_aliases={0: 1},
    )(tblH0, fbp, Wip, W_h)


def _table_update(tbl, binput, SH, W):
    def body(tbl_ref, bi_ref, sh_ref, w_ref, out_ref):
        g = jnp.maximum(bi_ref[...] + sh_ref[...], 0.0)
        out_ref[...] = jnp.dot(g, w_ref[...], precision=_PREC)

    nb = N_BONDS // BM
    off = N_MESS // BM
    return pl.pallas_call(
        body,
        grid=(nb,),
        in_specs=[
            pl.BlockSpec(memory_space=pl.ANY),
            pl.BlockSpec((BM, HIDDEN), lambda i: (i, 0)),
            pl.BlockSpec((BM, HIDDEN), lambda i: (i, 0)),
            pl.BlockSpec((HIDDEN, HIDDEN), lambda i: (0, 0)),
        ],
        out_specs=pl.BlockSpec((BM, HIDDEN), lambda i: (i + off, 0)),
        out_shape=jax.ShapeDtypeStruct((N_TBL, HIDDEN), jnp.float32),
        input_output_aliases={0: 0},
    )(tbl, binput, SH, W)


def _bond_final(tblO0, binput, SH, Wo2):
    def body(tbl_ref, bi_ref, sh_ref, w_ref, out_ref):
        g = jnp.maximum(bi_ref[...] + sh_ref[...], 0.0)
        out_ref[...] = jnp.dot(g, w_ref[...], precision=_PREC)

    nb = N_BONDS // BM
    off = N_MESS // BM
    return pl.pallas_call(
        body,
        grid=(nb,),
        in_specs=[
            pl.BlockSpec(memory_space=pl.ANY),
            pl.BlockSpec((BM, HIDDEN), lambda i: (i, 0)),
            pl.BlockSpec((BM, HIDDEN), lambda i: (i, 0)),
            pl.BlockSpec((HIDDEN, HIDDEN), lambda i: (0, 0)),
        ],
        out_specs=pl.BlockSpec((BM, HIDDEN), lambda i: (i + off, 0)),
        out_shape=jax.ShapeDtypeStruct((N_TBL, HIDDEN), jnp.float32),
        input_output_aliases={0: 0},
    )(tblO0, binput, SH, Wo2)


def _mol_head(fap, Wo1p, b2, nei):
    def body(fa_ref, w_ref, b_ref, nei_ref, out_ref):
        ah = jnp.dot(fa_ref[...], w_ref[...], precision=_PREC)
        ah = jnp.maximum(ah + b_ref[...] + nei_ref[...], 0.0)
        ah = ah.reshape(N_MOLS, ATOMS_PER_MOL, HIDDEN)
        out_ref[...] = jnp.mean(ah, axis=1)

    return pl.pallas_call(
        body,
        out_shape=jax.ShapeDtypeStruct((N_MOLS, HIDDEN), jnp.float32),
    )(fap, Wo1p, b2, nei)


def kernel(fatoms, fbonds, agraph, bgraph, tree_message, W_i, W_h, W_o, b_o):
    f32 = jnp.float32
    fbp = jnp.pad(fbonds.astype(f32), ((0, 0), (0, 64 - ATOM_FDIM - BOND_FDIM)))
    Wip = jnp.pad(W_i.astype(f32), ((0, 64 - ATOM_FDIM - BOND_FDIM), (0, 0)))
    fap = jnp.pad(fatoms.astype(f32), ((0, 0), (0, 64 - ATOM_FDIM)))
    Wo1p = jnp.pad(W_o[:ATOM_FDIM].astype(f32), ((0, 64 - ATOM_FDIM), (0, 0)))
    Wo2 = W_o[ATOM_FDIM:].astype(f32)
    b2 = b_o.astype(f32).reshape(1, HIDDEN)

    CH = 40
    bidx = _arrange_idx(bgraph, N_BONDS, CH)
    A_PAD = 10240
    ag = jnp.zeros((A_PAD, MAX_NB), jnp.int32).at[:N_ATOMS].set(
        agraph.astype(jnp.int32))
    aidx = _arrange_idx(ag, A_PAD, CH)

    tblH0, tblO0 = _tree_transform(tree_message.astype(f32), W_h.astype(f32), Wo2)
    binput, tbl = _bond_init(tblH0, fbp, Wip, W_h.astype(f32))

    for _ in range(DEPTH - 2):
        SH = _gather_sum_sc(tbl, bidx, N_BONDS, CH)
        tbl = _table_update(tbl, binput, SH, W_h.astype(f32))
    SH = _gather_sum_sc(tbl, bidx, N_BONDS, CH)
    tblO = _bond_final(tblO0, binput, SH, Wo2)

    neiO = _gather_sum_sc(tblO, aidx, A_PAD, CH)[:N_ATOMS]

    return _mol_head(fap, Wo1p, b2, neiO)


# speedup vs baseline: 1.0781x; 1.0781x over previous
import functools

import jax
import jax.numpy as jnp
from jax import lax
from jax.experimental import pallas as pl
from jax.experimental.pallas import tpu as pltpu
from jax.experimental.pallas import tpu_sc as plsc

HIDDEN = 128
DEPTH = 6
ATOM_FDIM = 35
BOND_FDIM = 5
MAX_NB = 10
N_ATOMS = 10000
N_BONDS = 320000
N_MESS = 4096
N_MOLS = 100
ATOMS_PER_MOL = 100
N_TBL = N_MESS + N_BONDS

NW = 32
LANES = 16
_PREC = jax.lax.Precision.DEFAULT

BM = 512


def _gather_sum_sc(tbl, idx_arr, B, CH):
    bpw = B // NW
    nch = bpw // CH
    assert nch % 2 == 0
    mesh = plsc.VectorSubcoreMesh(core_axis_name="c", subcore_axis_name="s")

    @functools.partial(
        pl.kernel,
        mesh=mesh,
        out_type=jax.ShapeDtypeStruct((B, HIDDEN), jnp.float32),
    scratch_types=[
            pltpu.VMEM((2, MAX_NB, CH), jnp.int32),
            pltpu.VMEM((2, MAX_NB, CH, HIDDEN), jnp.float32),
            pltpu.VMEM((2, CH, HIDDEN), jnp.float32),
            pltpu.SemaphoreType.DMA,
            pltpu.SemaphoreType.DMA,
            pltpu.SemaphoreType.DMA,
            pltpu.SemaphoreType.DMA,
        ],
    )
    def k(tbl_hbm, idx_hbm, out_hbm, idx_v, rows_v, acc_v, sem0, sem1, os0, os1):
        wid = lax.axis_index("s") * 2 + lax.axis_index("c")
        sems = (sem0, sem1)
        osems = (os0, os1)

        def fetch(c, buf):
            pltpu.sync_copy(idx_hbm.at[wid * nch + c], idx_v.at[buf])
            for r in range(MAX_NB):
                pltpu.async_copy(
                    tbl_hbm.at[idx_v.at[buf, r]], rows_v.at[buf, r], sems[buf])

        def drain(c, buf):
            for r in range(MAX_NB):
                pltpu.make_async_copy(
                    tbl_hbm.at[idx_v.at[buf, r]], rows_v.at[buf, r],
                    sems[buf]).wait()

        def out_slice(c):
            return out_hbm.at[pl.ds(wid * bpw + c * CH, CH)]

        def accum(c, buf):
            @plsc.parallel_loop(0, CH, 1, unroll=2)
            def seg(i):
                for j in range(HIDDEN // LANES):
                    sl = pl.ds(LANES * j, LANES)
                    s = rows_v[buf, 0, i, sl]
                    for r in range(1, MAX_NB):
                        s = s + rows_v[buf, r, i, sl]
                    acc_v[buf, i, sl] = s

            pltpu.async_copy(acc_v.at[buf], out_slice(c), osems[buf])

        def wait_store(c, buf):
            pltpu.make_async_copy(acc_v.at[buf], out_slice(c), osems[buf]).wait()

        fetch(0, 0)

        def body(i, carry):
            c0 = 2 * i
            fetch(c0 + 1, 1)
            drain(c0, 0)

            @pl.when(i > 0)
            def _():
                wait_store(c0 - 2, 0)

            accum(c0, 0)

            @pl.when(c0 + 2 < nch)
            def _():
                fetch(c0 + 2, 0)

            drain(c0 + 1, 1)

            @pl.when(i > 0)
            def _():
                wait_store(c0 - 1, 1)

            accum(c0 + 1, 1)
            return carry

        lax.fori_loop(0, nch // 2, body, 0)
        wait_store(nch - 2, 0)
        wait_store(nch - 1, 1)

    return k(tbl, idx_arr)


def _arrange_idx(g, B, CH):
    nch = B // NW // CH
    g = g.astype(jnp.int32).reshape(NW * nch, CH, MAX_NB)
    return jnp.transpose(g, (0, 2, 1))


def _tree_transform(tree, W_h, Wo2):
    def body(t_ref, wh_ref, wo_ref, th_ref, to_ref):
        t = t_ref[...]
        th_ref[...] = jnp.dot(t, wh_ref[...], precision=_PREC)
        to_ref[...] = jnp.dot(t, wo_ref[...], precision=_PREC)

    nb = N_MESS // BM
    return pl.pallas_call(
        body,
        grid=(nb,),
        in_specs=[
            pl.BlockSpec((BM, HIDDEN), lambda i: (i, 0)),
            pl.BlockSpec((HIDDEN, HIDDEN), lambda i: (0, 0)),
            pl.BlockSpec((HIDDEN, HIDDEN), lambda i: (0, 0)),
        ],
        out_specs=(
            pl.BlockSpec((BM, HIDDEN), lambda i: (i, 0)),
            pl.BlockSpec((BM, HIDDEN), lambda i: (i, 0)),
        ),
        out_shape=(
            jax.ShapeDtypeStruct((N_TBL, HIDDEN), jnp.float32),
            jax.ShapeDtypeStruct((N_TBL, HIDDEN), jnp.float32),
        ),
    )(tree, W_h, Wo2)


def _bond_init(tblH0, fbp, Wip, W_h):
    def body(tbl_ref, fb_ref, wi_ref, wh_ref, bi_ref, gh_ref):
        bi = jnp.dot(fb_ref[...], wi_ref[...], precision=_PREC)
        bi_ref[...] = bi
        gh_ref[...] = jnp.dot(jnp.maximum(bi, 0.0), wh_ref[...], precision=_PREC)

    nb = N_BONDS // BM
    off = N_MESS // BM
    return pl.pallas_call(
        body,
        grid=(nb,),
        in_specs=[
            pl.BlockSpec(memory_space=pl.ANY),
            pl.BlockSpec((BM, 64), lambda i: (i, 0)),
            pl.BlockSpec((64, HIDDEN), lambda i: (0, 0)),
            pl.BlockSpec((HIDDEN, HIDDEN), lambda i: (0, 0)),
        ],
        out_specs=(
            pl.BlockSpec((BM, HIDDEN), lambda i: (i, 0)),
            pl.BlockSpec((BM, HIDDEN), lambda i: (i + off, 0)),
        ),
        out_shape=(
            jax.ShapeDtypeStruct((N_BONDS, HIDDEN), jnp.float32),
            jax.ShapeDtypeStruct((N_TBL, HIDDEN), jnp.float32),
        ),
        input_output_aliases={0: 1},
    )(tblH0, fbp, Wip, W_h)


def _table_update(tbl, binput, SH, W):
    def body(tbl_ref, bi_ref, sh_ref, w_ref, out_ref):
        g = jnp.maximum(bi_ref[...] + sh_ref[...], 0.0)
        out_ref[...] = jnp.dot(g, w_ref[...], precision=_PREC)

    nb = N_BONDS // BM
    off = N_MESS // BM
    return pl.pallas_call(
        body,
        grid=(nb,),
        in_specs=[
            pl.BlockSpec(memory_space=pl.ANY),
            pl.BlockSpec((BM, HIDDEN), lambda i: (i, 0)),
            pl.BlockSpec((BM, HIDDEN), lambda i: (i, 0)),
            pl.BlockSpec((HIDDEN, HIDDEN), lambda i: (0, 0)),
        ],
        out_specs=pl.BlockSpec((BM, HIDDEN), lambda i: (i + off, 0)),
        out_shape=jax.ShapeDtypeStruct((N_TBL, HIDDEN), jnp.float32),
        input_output_aliases={0: 0},
    )(tbl, binput, SH, W)


def _bond_final(tblO0, binput, SH, Wo2):
    def body(tbl_ref, bi_ref, sh_ref, w_ref, out_ref):
        g = jnp.maximum(bi_ref[...] + sh_ref[...], 0.0)
        out_ref[...] = jnp.dot(g, w_ref[...], precision=_PREC)

    nb = N_BONDS // BM
    off = N_MESS // BM
    return pl.pallas_call(
        body,
        grid=(nb,),
        in_specs=[
            pl.BlockSpec(memory_space=pl.ANY),
            pl.BlockSpec((BM, HIDDEN), lambda i: (i, 0)),
            pl.BlockSpec((BM, HIDDEN), lambda i: (i, 0)),
            pl.BlockSpec((HIDDEN, HIDDEN), lambda i: (0, 0)),
        ],
        out_specs=pl.BlockSpec((BM, HIDDEN), lambda i: (i + off, 0)),
        out_shape=jax.ShapeDtypeStruct((N_TBL, HIDDEN), jnp.float32),
        input_output_aliases={0: 0},
    )(tblO0, binput, SH, Wo2)


def _mol_head(fap, Wo1p, b2, nei):
    def body(fa_ref, w_ref, b_ref, nei_ref, out_ref):
        ah = jnp.dot(fa_ref[...], w_ref[...], precision=_PREC)
        ah = jnp.maximum(ah + b_ref[...] + nei_ref[...], 0.0)
        ah = ah.reshape(N_MOLS, ATOMS_PER_MOL, HIDDEN)
        out_ref[...] = jnp.mean(ah, axis=1)

    return pl.pallas_call(
        body,
        out_shape=jax.ShapeDtypeStruct((N_MOLS, HIDDEN), jnp.float32),
    )(fap, Wo1p, b2, nei)


def kernel(fatoms, fbonds, agraph, bgraph, tree_message, W_i, W_h, W_o, b_o):
    f32 = jnp.float32
    fbp = jnp.pad(fbonds.astype(f32), ((0, 0), (0, 64 - ATOM_FDIM - BOND_FDIM)))
    Wip = jnp.pad(W_i.astype(f32), ((0, 64 - ATOM_FDIM - BOND_FDIM), (0, 0)))
    fap = jnp.pad(fatoms.astype(f32), ((0, 0), (0, 64 - ATOM_FDIM)))
    Wo1p = jnp.pad(W_o[:ATOM_FDIM].astype(f32), ((0, 64 - ATOM_FDIM), (0, 0)))
    Wo2 = W_o[ATOM_FDIM:].astype(f32)
    b2 = b_o.astype(f32).reshape(1, HIDDEN)

    CH = 40
    bidx = _arrange_idx(bgraph, N_BONDS, CH)
    A_PAD = 10240
    ag = jnp.zeros((A_PAD, MAX_NB), jnp.int32).at[:N_ATOMS].set(
        agraph.astype(jnp.int32))
    aidx = _arrange_idx(ag, A_PAD, CH)

    tblH0, tblO0 = _tree_transform(tree_message.astype(f32), W_h.astype(f32), Wo2)
    binput, tbl = _bond_init(tblH0, fbp, Wip, W_h.astype(f32))

    for _ in range(DEPTH - 2):
        SH = _gather_sum_sc(tbl, bidx, N_BONDS, CH)
        tbl = _table_update(tbl, binput, SH, W_h.astype(f32))
    SH = _gather_sum_sc(tbl, bidx, N_BONDS, CH)
    tblO = _bond_final(tblO0, binput, SH, Wo2)

    neiO = _gather_sum_sc(tblO, aidx, A_PAD, CH)[:N_ATOMS]

    return _mol_head(fap, Wo1p, b2, neiO)
